# baseline (speedup 1.0000x reference)
import functools

import jax
import jax.numpy as jnp
from jax import lax
from jax.experimental import pallas as pl
from jax.experimental.pallas import tpu as pltpu
from jax.experimental.pallas import tpu_sc as plsc

NC = 2
NS = 16
NW = NC * NS
L = 16
IW = 128


@functools.lru_cache(maxsize=None)
def _make(T, B, D):
    BW = B // NW
    KG = BW // IW
    mesh = plsc.VectorSubcoreMesh(
        core_axis_name="c", subcore_axis_name="s",
        num_cores=NC, num_subcores=NS)

    @functools.partial(
        pl.kernel,
        out_type=jax.ShapeDtypeStruct((T, D, B), jnp.float32),
        mesh=mesh,
        scratch_types=[
            pltpu.VMEM((KG, IW), jnp.int32),
            pltpu.VMEM((BW, D), jnp.float32),
            pltpu.VMEM((D, BW), jnp.float32),
            pltpu.SemaphoreType.DMA,
        ],
        compiler_params=pltpu.CompilerParams(
            use_tc_tiling_on_sc=False, needs_layout_passes=False),
    )
    def k(ids_hbm, table_hbm, y_hbm, idx_v, rows_v, tr_v, sem):
        wid = lax.axis_index("s") * NC + lax.axis_index("c")
        b0 = wid * BW
        lane = lax.iota(jnp.int32, L)

        def step(t, carry):
            pltpu.sync_copy(ids_hbm.at[t, pl.ds(wid * KG, KG)], idx_v)
            cps = [
                pltpu.async_copy(table_hbm.at[idx_v.at[j]],
                                 rows_v.at[pl.ds(j * IW, IW)], sem)
                for j in range(KG)
            ]
            for cp in cps:
                cp.wait()

            def tr_step(g, c2):
                row = g * L + lane
                for d in range(D):
                    col = jnp.full((L,), d, jnp.int32)
                    tr_v[d, pl.ds(g * L, L)] = plsc.load_gather(
                        rows_v, [row, col])
                return c2

            lax.fori_loop(0, BW // L, tr_step, 0)
            pltpu.sync_copy(tr_v, y_hbm.at[t, :, pl.ds(b0, BW)])
            return carry

        lax.fori_loop(0, T, step, 0)

    return k


def kernel(object_ids, table):
    S, T = object_ids.shape
    D = table.shape[1]
    ids_t = object_ids.T.astype(jnp.int32).reshape(T, S // IW, IW)
    y = _make(T, S, D)(ids_t, table)
    return jnp.transpose(y, (2, 0, 1))


# speedup vs baseline: 1.3690x; 1.3690x over previous
import functools

import jax
import jax.numpy as jnp
from jax import lax
from jax.experimental import pallas as pl
from jax.experimental.pallas import tpu as pltpu
from jax.experimental.pallas import tpu_sc as plsc

NC = 2
NS = 16
NW = NC * NS
L = 16
IW = 128


@functools.lru_cache(maxsize=None)
def _make(T, B, D):
    BW = B // NW
    KG = BW // IW
    mesh = plsc.VectorSubcoreMesh(
        core_axis_name="c", subcore_axis_name="s",
        num_cores=NC, num_subcores=NS)

    @functools.partial(
        pl.kernel,
        out_type=jax.ShapeDtypeStruct((T, D, B), jnp.float32),
        mesh=mesh,
        scratch_types=[
            pltpu.VMEM((KG, IW), jnp.int32),
            pltpu.VMEM((BW, D), jnp.float32),
            pltpu.VMEM((D, BW), jnp.float32),
            pltpu.SemaphoreType.DMA,
        ],
        compiler_params=pltpu.CompilerParams(
            use_tc_tiling_on_sc=False, needs_layout_passes=False),
    )
    def k(ids_hbm, table_hbm, y_hbm, idx_v, rows_v, tr_v, sem):
        wid = lax.axis_index("s") * NC + lax.axis_index("c")
        b0 = wid * BW
        lane = lax.iota(jnp.int32, L)
        cols = [jnp.full((L,), d, jnp.int32) for d in range(D)]

        def step(t, carry):
            pltpu.sync_copy(ids_hbm.at[t, pl.ds(wid * KG, KG)], idx_v)
            cps = [
                pltpu.async_copy(table_hbm.at[idx_v.at[j]],
                                 rows_v.at[pl.ds(j * IW, IW)], sem)
                for j in range(KG)
            ]
            for cp in cps:
                cp.wait()

            @plsc.parallel_loop(0, BW // L, 1, unroll=2)
            def tr_step(g):
                row = g * L + lane
                for d in range(D):
                    tr_v[d, pl.ds(g * L, L)] = plsc.load_gather(
                        rows_v, [row, cols[d]])

            pltpu.sync_copy(tr_v, y_hbm.at[t, :, pl.ds(b0, BW)])
            return carry

        lax.fori_loop(0, T, step, 0)

    return k


def kernel(object_ids, table):
    S, T = object_ids.shape
    D = table.shape[1]
    ids_t = object_ids.T.astype(jnp.int32).reshape(T, S // IW, IW)
    y = _make(T, S, D)(ids_t, table)
    return jnp.transpose(y, (2, 0, 1))


# speedup vs baseline: 1.5464x; 1.1296x over previous
import functools

import jax
import jax.numpy as jnp
from jax import lax
from jax.experimental import pallas as pl
from jax.experimental.pallas import tpu as pltpu
from jax.experimental.pallas import tpu_sc as plsc

NC = 2
NS = 16
NW = NC * NS
L = 16
IW = 128


@functools.lru_cache(maxsize=None)
def _make(T, B, D):
    BW = B // NW
    KG = BW // IW
    mesh = plsc.VectorSubcoreMesh(
        core_axis_name="c", subcore_axis_name="s",
        num_cores=NC, num_subcores=NS)

    scratch = [
        pltpu.VMEM((KG, IW), jnp.int32),
        pltpu.VMEM((KG, IW), jnp.int32),
        pltpu.VMEM((BW, D), jnp.float32),
        pltpu.VMEM((BW, D), jnp.float32),
        pltpu.VMEM((D, BW), jnp.float32),
        pltpu.SemaphoreType.DMA,
        pltpu.SemaphoreType.DMA,
    ]

    @functools.partial(
        pl.kernel,
        out_type=jax.ShapeDtypeStruct((T, D, B), jnp.float32),
        mesh=mesh,
        scratch_types=scratch,
        compiler_params=pltpu.CompilerParams(
            use_tc_tiling_on_sc=False, needs_layout_passes=False),
    )
    def k(ids_hbm, table_hbm, y_hbm, idx_a, idx_b, rows_a, rows_b, tr_v,
          sem_a, sem_b):
        wid = lax.axis_index("s") * NC + lax.axis_index("c")
        b0 = wid * BW
        lane = lax.iota(jnp.int32, L)
        cols = [jnp.full((L,), d, jnp.int32) for d in range(D)]

        def load_idx(t, idx_ref):
            pltpu.sync_copy(ids_hbm.at[t, pl.ds(wid * KG, KG)], idx_ref)

        def fire(idx_ref, rows_ref, sem):
            for j in range(KG):
                pltpu.async_copy(table_hbm.at[idx_ref.at[j]],
                                 rows_ref.at[pl.ds(j * IW, IW)], sem)

        def drain(rows_ref, sem):
            pltpu.make_async_copy(table_hbm.at[pl.ds(0, BW)], rows_ref,
                                  sem).wait()

        def transpose_store(t, rows_ref):
            @plsc.parallel_loop(0, BW // L, 1, unroll=2)
            def tr_step(g):
                row = g * L + lane
                for d in range(D):
                    tr_v[d, pl.ds(g * L, L)] = plsc.load_gather(
                        rows_ref, [row, cols[d]])

            pltpu.sync_copy(tr_v, y_hbm.at[t, :, pl.ds(b0, BW)])

        load_idx(0, idx_a)
        fire(idx_a, rows_a, sem_a)

        def step2(i, carry):
            t0 = 2 * i
            load_idx(t0 + 1, idx_b)
            fire(idx_b, rows_b, sem_b)
            drain(rows_a, sem_a)
            transpose_store(t0, rows_a)

            @pl.when(t0 + 2 < T)
            def _():
                load_idx(t0 + 2, idx_a)
                fire(idx_a, rows_a, sem_a)

            drain(rows_b, sem_b)
            transpose_store(t0 + 1, rows_b)
            return carry

        lax.fori_loop(0, T // 2, step2, 0)

    return k


def kernel(object_ids, table):
    S, T = object_ids.shape
    D = table.shape[1]
    ids_t = object_ids.T.astype(jnp.int32).reshape(T, S // IW, IW)
    y = _make(T, S, D)(ids_t, table)
    return jnp.transpose(y, (2, 0, 1))
